# Initial kernel scaffold; baseline (speedup 1.0000x reference)
import jax
import jax.numpy as jnp
from jax.experimental import pallas as pl


def kernel(inputs, tables, gamma, beta, W1, b1, W2, b2, W3, b3, Wo, bo):
    raise NotImplementedError("write your pallas kernel here")


# speedup vs baseline: 1.0422x; 1.0422x over previous
import functools

import jax
import jax.numpy as jnp
from jax import lax
from jax.experimental import pallas as pl
from jax.experimental.pallas import tpu as pltpu
from jax.experimental.pallas import tpu_sc as plsc

_B = 4096
_F = 26
_V = 100000
_D = 16
_EPS = 1e-3

_NC = 2
_NS = 16
_NW = _NC * _NS
_BPW = _B // _NW
_IPW = _BPW * _F
_NVEC = _IPW // 16


def _sc_body(tbl_hbm, idx_hbm, off_hbm, bi_hbm, idx_v, off_v, flat_v,
             rows_v, bi_v, sem):
    wid = lax.axis_index("s") * _NC + lax.axis_index("c")
    base_i = wid * _IPW
    base_b = wid * _BPW

    pltpu.sync_copy(idx_hbm.at[pl.ds(base_i, _IPW)], idx_v)
    pltpu.sync_copy(off_hbm, off_v)

    def add_body(j, carry):
        o = pl.multiple_of(j * 16, 16)
        flat_v[pl.ds(o, 16)] = idx_v[pl.ds(o, 16)] + off_v[pl.ds(o, 16)]
        return carry

    lax.fori_loop(0, _NVEC, add_body, 0)

    descs = []
    for c in range(_F):
        descs.append(pltpu.async_copy(
            tbl_hbm.at[flat_v.at[pl.ds(c * 128, 128)]],
            rows_v.at[pl.ds(c * 128, 128)],
            sem,
        ))
    for d in descs:
        d.wait()

    def pool_body(b, carry):
        i0 = b * _F
        e = rows_v[i0, :]
        s = e
        sq = e * e
        for f in range(1, _F):
            e = rows_v[i0 + f, :]
            s = s + e
            sq = sq + e * e
        bi_v[b, :] = 0.5 * (s * s - sq)
        return carry

    lax.fori_loop(0, _BPW, pool_body, 0)

    pltpu.sync_copy(bi_v, bi_hbm.at[pl.ds(base_b, _BPW)])


@jax.jit
def _sc_gather_pool(tbl, idx, off):
    mesh = plsc.VectorSubcoreMesh(core_axis_name="c", subcore_axis_name="s")
    return pl.kernel(
        _sc_body,
        out_type=jax.ShapeDtypeStruct((_B, _D), jnp.float32),
        mesh=mesh,
        scratch_types=[
            pltpu.VMEM((_IPW,), jnp.int32),
            pltpu.VMEM((_IPW,), jnp.int32),
            pltpu.VMEM((_IPW,), jnp.int32),
            pltpu.VMEM((_IPW, _D), jnp.float32),
            pltpu.VMEM((_BPW, _D), jnp.float32),
            pltpu.SemaphoreType.DMA,
        ],
        compiler_params=pltpu.CompilerParams(use_tc_tiling_on_sc=False),
    )(tbl, idx, off)


def _tc_body(bi_ref, gamma_ref, beta_ref, W1_ref, b1_ref, W2_ref, b2_ref,
             W3_ref, b3_ref, Wo_ref, bo_ref, out_ref):
    bi = bi_ref[...]
    mean = jnp.mean(bi, axis=0, keepdims=True)
    var = jnp.mean((bi - mean) ** 2, axis=0, keepdims=True)
    x = (bi - mean) * lax.rsqrt(var + _EPS) * gamma_ref[...] + beta_ref[...]
    x = jnp.maximum(jnp.dot(x, W1_ref[...],
                            preferred_element_type=jnp.float32)
                    + b1_ref[...], 0.0)
    x = jnp.maximum(jnp.dot(x, W2_ref[...],
                            preferred_element_type=jnp.float32)
                    + b2_ref[...], 0.0)
    x = jnp.maximum(jnp.dot(x, W3_ref[...],
                            preferred_element_type=jnp.float32)
                    + b3_ref[...], 0.0)
    z = jnp.dot(x, Wo_ref[...], preferred_element_type=jnp.float32) \
        + bo_ref[...]
    out_ref[...] = 1.0 / (1.0 + jnp.exp(-z))


@jax.jit
def _tc_bn_mlp(bi, gamma, beta, W1, b1, W2, b2, W3, b3, Wo, bo):
    return pl.pallas_call(
        _tc_body,
        out_shape=jax.ShapeDtypeStruct((_B, 1), jnp.float32),
    )(bi, gamma, beta, W1, b1, W2, b2, W3, b3, Wo, bo)


def kernel(inputs, tables, gamma, beta, W1, b1, W2, b2, W3, b3, Wo, bo):
    tbl = tables.reshape(_F * _V, _D)
    idx = inputs.reshape(_B * _F)
    off = jnp.tile(jnp.arange(_F, dtype=jnp.int32) * _V, _BPW)
    bi = _sc_gather_pool(tbl, idx, off)
    return _tc_bn_mlp(
        bi, gamma.reshape(1, _D), beta.reshape(1, _D),
        W1, b1.reshape(1, -1), W2, b2.reshape(1, -1),
        W3, b3.reshape(1, -1), Wo, bo.reshape(1, 1),
    )


# speedup vs baseline: 1.0465x; 1.0041x over previous
import jax
import jax.numpy as jnp
from jax import lax
from jax.experimental import pallas as pl
from jax.experimental.pallas import tpu as pltpu
from jax.experimental.pallas import tpu_sc as plsc

_B = 4096
_F = 26
_V = 100000
_D = 16
_EPS = 1e-3

_NC = 2
_NS = 16
_NW = _NC * _NS
_BPW = _B // _NW


def _sc_body(tbl_hbm, idx_hbm, bi_hbm, idx_v, rows_v, bi_v, sem):
    wid = lax.axis_index("s") * _NC + lax.axis_index("c")
    base_b = wid * _BPW

    pltpu.sync_copy(idx_hbm.at[:, pl.ds(base_b, _BPW)], idx_v)

    descs = []
    for f in range(_F):
        descs.append(pltpu.async_copy(
            tbl_hbm.at[f].at[idx_v.at[f]],
            rows_v.at[pl.ds(f * _BPW, _BPW)],
            sem,
        ))
    for d in descs:
        d.wait()

    def pool_body(b, carry):
        e = rows_v[b, :]
        s = e
        sq = e * e
        for f in range(1, _F):
            e = rows_v[f * _BPW + b, :]
            s = s + e
            sq = sq + e * e
        bi_v[b, :] = 0.5 * (s * s - sq)
        return carry

    lax.fori_loop(0, _BPW, pool_body, 0)

    pltpu.sync_copy(bi_v, bi_hbm.at[pl.ds(base_b, _BPW)])


@jax.jit
def _sc_gather_pool(tbl, idxT):
    mesh = plsc.VectorSubcoreMesh(core_axis_name="c", subcore_axis_name="s")
    return pl.kernel(
        _sc_body,
        out_type=jax.ShapeDtypeStruct((_B, _D), jnp.float32),
        mesh=mesh,
        scratch_types=[
            pltpu.VMEM((_F, _BPW), jnp.int32),
            pltpu.VMEM((_F * _BPW, _D), jnp.float32),
            pltpu.VMEM((_BPW, _D), jnp.float32),
            pltpu.SemaphoreType.DMA,
        ],
        compiler_params=pltpu.CompilerParams(use_tc_tiling_on_sc=False),
    )(tbl, idxT)


def _tc_body(bi_ref, gamma_ref, beta_ref, W1_ref, b1_ref, W2_ref, b2_ref,
             W3_ref, b3_ref, Wo_ref, bo_ref, out_ref):
    bi = bi_ref[...]
    mean = jnp.mean(bi, axis=0, keepdims=True)
    var = jnp.mean((bi - mean) ** 2, axis=0, keepdims=True)
    x = (bi - mean) * lax.rsqrt(var + _EPS) * gamma_ref[...] + beta_ref[...]
    x = jnp.maximum(jnp.dot(x, W1_ref[...],
                            preferred_element_type=jnp.float32)
                    + b1_ref[...], 0.0)
    x = jnp.maximum(jnp.dot(x, W2_ref[...],
                            preferred_element_type=jnp.float32)
                    + b2_ref[...], 0.0)
    x = jnp.maximum(jnp.dot(x, W3_ref[...],
                            preferred_element_type=jnp.float32)
                    + b3_ref[...], 0.0)
    z = jnp.dot(x, Wo_ref[...], preferred_element_type=jnp.float32) \
        + bo_ref[...]
    out_ref[...] = 1.0 / (1.0 + jnp.exp(-z))


@jax.jit
def _tc_bn_mlp(bi, gamma, beta, W1, b1, W2, b2, W3, b3, Wo, bo):
    return pl.pallas_call(
        _tc_body,
        out_shape=jax.ShapeDtypeStruct((_B, 1), jnp.float32),
    )(bi, gamma, beta, W1, b1, W2, b2, W3, b3, Wo, bo)


def kernel(inputs, tables, gamma, beta, W1, b1, W2, b2, W3, b3, Wo, bo):
    idxT = inputs.T
    bi = _sc_gather_pool(tables, idxT)
    return _tc_bn_mlp(
        bi, gamma.reshape(1, _D), beta.reshape(1, _D),
        W1, b1.reshape(1, -1), W2, b2.reshape(1, -1),
        W3, b3.reshape(1, -1), Wo, bo.reshape(1, 1),
    )
